# baseline (speedup 1.0000x reference)
import numpy as np
import jax
import jax.numpy as jnp
from jax.experimental import pallas as pl
from jax.experimental.pallas import tpu as pltpu

_E, _B, _S, _C = 10, 128, 4000, 5
_SITES = 128
_LANES = _SITES * _C
_NROWS = _B * _S // _SITES
_RB = 80
_CH = 80


def _sel5(masks, parts):
    return jnp.where(masks[0], parts[0],
           jnp.where(masks[1], parts[1],
           jnp.where(masks[2], parts[2],
           jnp.where(masks[3], parts[3], parts[4]))))


def _vote_kernel(x_ref, n_ref, o_ref):
    lane = jax.lax.broadcasted_iota(jnp.int32, (_CH, _SITES), 1)
    site = lane
    qmask = [[(2 * (c - lane)) % _C == q for q in range(4)] for c in range(_C)]
    cmask = [[(3 * q + lane) % _C == c for c in range(4)] for q in range(_C)]
    idx_in = [(_C * site + c) % _SITES for c in range(_C)]
    idx_out = [(_SITES * ((2 * (c - lane)) % _C) + lane) // _C for c in range(_C)]

    for rr in range(_RB // _CH):
        r0 = rr * _CH
        acc = [None] * _C
        for e in range(_E):
            xe = x_ref[e, r0:r0 + _CH, :]
            xq = [xe[:, _SITES * q:_SITES * (q + 1)] for q in range(_C)]
            w = 1.0 + n_ref[e, r0:r0 + _CH, :] * 0.001
            xc = []
            for c in range(_C):
                mid = _sel5(qmask[c], xq)
                xc.append(jnp.take_along_axis(mid, idx_in[c], axis=-1))
            m01 = jnp.maximum(xc[0], xc[1])
            m23 = jnp.maximum(xc[2], xc[3])
            m = jnp.maximum(jnp.maximum(m01, m23), xc[4])
            for c in range(_C):
                contrib = jnp.where(xc[c] == m, w, 0.0)
                acc[c] = contrib if acc[c] is None else acc[c] + contrib
        accp = [jnp.take_along_axis(acc[c], idx_out[c], axis=-1)
                for c in range(_C)]
        for q in range(_C):
            o_ref[r0:r0 + _CH, _SITES * q:_SITES * (q + 1)] = _sel5(cmask[q], accp)


def kernel(expert_logits, noise):
    E, B, S, C = expert_logits.shape
    x = expert_logits.reshape(E, _NROWS, _LANES)
    nz = noise.reshape(E, _NROWS, _SITES)
    out = pl.pallas_call(
        _vote_kernel,
        grid=(_NROWS // _RB,),
        in_specs=[
            pl.BlockSpec((E, _RB, _LANES), lambda i: (0, i, 0)),
            pl.BlockSpec((E, _RB, _SITES), lambda i: (0, i, 0)),
        ],
        out_specs=pl.BlockSpec((_RB, _LANES), lambda i: (i, 0)),
        out_shape=jax.ShapeDtypeStruct((_NROWS, _LANES), expert_logits.dtype),
        compiler_params=pltpu.CompilerParams(
            dimension_semantics=("arbitrary",),
        ),
    )(x, nz)
    return out.reshape(B, S, C)


# speedup vs baseline: 61.0192x; 61.0192x over previous
import jax
import jax.numpy as jnp
from jax.experimental import pallas as pl
from jax.experimental.pallas import tpu as pltpu

_E, _B, _S, _C = 10, 128, 4000, 5
_SB = 200
_CH = 40


def _vote_kernel(x_ref, n_ref, o_ref):
    for rr in range(_SB // _CH):
        r0 = rr * _CH
        acc = [None] * _C
        for e in range(_E):
            xc = [x_ref[e, c, r0:r0 + _CH, :] for c in range(_C)]
            w = 1.0 + n_ref[e, r0:r0 + _CH, :] * 0.001
            m01 = jnp.maximum(xc[0], xc[1])
            m23 = jnp.maximum(xc[2], xc[3])
            m = jnp.maximum(jnp.maximum(m01, m23), xc[4])
            for c in range(_C):
                contrib = jnp.where(xc[c] == m, w, 0.0)
                acc[c] = contrib if acc[c] is None else acc[c] + contrib
        for c in range(_C):
            o_ref[c, r0:r0 + _CH, :] = acc[c]


def kernel(expert_logits, noise):
    E, B, S, C = expert_logits.shape
    xt = jnp.transpose(expert_logits, (0, 3, 2, 1))
    nt = jnp.transpose(noise, (0, 2, 1))
    out = pl.pallas_call(
        _vote_kernel,
        grid=(_S // _SB,),
        in_specs=[
            pl.BlockSpec((E, C, _SB, B), lambda i: (0, 0, i, 0)),
            pl.BlockSpec((E, _SB, B), lambda i: (0, i, 0)),
        ],
        out_specs=pl.BlockSpec((C, _SB, B), lambda i: (0, i, 0)),
        out_shape=jax.ShapeDtypeStruct((C, S, B), expert_logits.dtype),
        compiler_params=pltpu.CompilerParams(
            dimension_semantics=("arbitrary",),
        ),
    )(xt, nt)
    return jnp.transpose(out, (2, 1, 0))


# speedup vs baseline: 61.3076x; 1.0047x over previous
import jax
import jax.numpy as jnp
from jax.experimental import pallas as pl
from jax.experimental.pallas import tpu as pltpu

_E, _B, _S, _C = 10, 128, 4000, 5
_SB = 400
_CH = 40


def _vote_kernel(x_ref, n_ref, o_ref):
    for rr in range(_SB // _CH):
        r0 = rr * _CH
        acc = [None] * _C
        for e in range(_E):
            xc = [x_ref[e, c, r0:r0 + _CH, :] for c in range(_C)]
            w = 1.0 + n_ref[e, r0:r0 + _CH, :] * 0.001
            m01 = jnp.maximum(xc[0], xc[1])
            m23 = jnp.maximum(xc[2], xc[3])
            m = jnp.maximum(jnp.maximum(m01, m23), xc[4])
            for c in range(_C):
                contrib = jnp.where(xc[c] == m, w, 0.0)
                acc[c] = contrib if acc[c] is None else acc[c] + contrib
        for c in range(_C):
            o_ref[c, r0:r0 + _CH, :] = acc[c]


def kernel(expert_logits, noise):
    E, B, S, C = expert_logits.shape
    xt = jnp.transpose(expert_logits, (0, 3, 2, 1))
    nt = jnp.transpose(noise, (0, 2, 1))
    out = pl.pallas_call(
        _vote_kernel,
        grid=(_S // _SB,),
        in_specs=[
            pl.BlockSpec((E, C, _SB, B), lambda i: (0, 0, i, 0)),
            pl.BlockSpec((E, _SB, B), lambda i: (0, i, 0)),
        ],
        out_specs=pl.BlockSpec((C, _SB, B), lambda i: (0, i, 0)),
        out_shape=jax.ShapeDtypeStruct((C, S, B), expert_logits.dtype),
        compiler_params=pltpu.CompilerParams(
            dimension_semantics=("arbitrary",),
        ),
    )(xt, nt)
    return jnp.transpose(out, (2, 1, 0))
